# Initial kernel scaffold; baseline (speedup 1.0000x reference)
import jax
import jax.numpy as jnp
from jax.experimental import pallas as pl


def kernel(x_cat, tables):
    raise NotImplementedError("write your pallas kernel here")


# speedup vs baseline: 1.0455x; 1.0455x over previous
import jax
import jax.numpy as jnp
from jax import lax
from jax.experimental import pallas as pl
from jax.experimental.pallas import tpu as pltpu
from jax.experimental.pallas import tpu_sc as plsc

_NUM_FIELDS = 26
_VOCAB = 100000
_D = 64
_B = 16384
_NC = 2
_NS = 16
_NW = _NC * _NS
_RPW = _B // _NW
_CHUNK = 128
_NCHUNK = _RPW // _CHUNK
_LANES = 16
_VPR = _D // _LANES


def _body(xt_hbm, tab_hbm, out_hbm, idx_v, buf_v, acc_v, sem0, sem1):
    wid = lax.axis_index("s") * _NC + lax.axis_index("c")
    base = wid * _RPW

    pltpu.sync_copy(xt_hbm.at[:, wid], idx_v)

    for f in range(1, _NUM_FIELDS):
        off = jnp.full((_LANES,), f * _VOCAB, dtype=jnp.int32)

        def _off_body(c, _, f=f, off=off):
            for v in range(_CHUNK // _LANES):
                plsc.addupdate(idx_v.at[f, c, pl.ds(v * _LANES, _LANES)], off)
            return 0

        lax.fori_loop(0, _NCHUNK, _off_body, 0)

    sems = (sem0, sem1)

    def _fire(f):
        p = f % 2
        return [
            pltpu.async_copy(
                tab_hbm.at[idx_v.at[f, c]],
                buf_v.at[p, pl.ds(c * _CHUNK, _CHUNK)],
                sems[p],
            )
            for c in range(_NCHUNK)
        ]

    handles = _fire(0)
    for f in range(_NUM_FIELDS):
        p = f % 2
        nxt = _fire(f + 1) if f + 1 < _NUM_FIELDS else None
        for h in handles:
            h.wait()
        handles = nxt

        if f == 0:
            def _init_body(r, _, p=p):
                for v in range(_VPR):
                    sl = pl.ds(v * _LANES, _LANES)
                    acc_v[r, sl] = buf_v[p, r, sl]
                return 0

            lax.fori_loop(0, _RPW, _init_body, 0)
        else:
            def _acc_body(r, _, p=p):
                for v in range(_VPR):
                    sl = pl.ds(v * _LANES, _LANES)
                    plsc.addupdate(acc_v.at[r, sl], buf_v[p, r, sl])
                return 0

            lax.fori_loop(0, _RPW, _acc_body, 0)

    pltpu.sync_copy(acc_v, out_hbm.at[pl.ds(base, _RPW)])


@jax.jit
def _run(xt4, flat_tab):
    mesh = plsc.VectorSubcoreMesh(core_axis_name="c", subcore_axis_name="s")
    f = pl.kernel(
        _body,
        out_type=jax.ShapeDtypeStruct((_B, _D), jnp.float32),
        mesh=mesh,
        compiler_params=pltpu.CompilerParams(use_tc_tiling_on_sc=False),
        scratch_types=[
            pltpu.VMEM((_NUM_FIELDS, _NCHUNK, _CHUNK), jnp.int32),
            pltpu.VMEM((2, _RPW, _D), jnp.float32),
            pltpu.VMEM((_RPW, _D), jnp.float32),
            pltpu.SemaphoreType.DMA,
            pltpu.SemaphoreType.DMA,
        ],
    )
    return f(xt4, flat_tab)


def kernel(x_cat, tables):
    xt4 = x_cat.T.reshape(_NUM_FIELDS, _NW, _NCHUNK, _CHUNK)
    flat = tables.reshape(_NUM_FIELDS * _VOCAB, _D)
    return _run(xt4, flat)
